# baseline (speedup 1.0000x reference)
import functools

import jax
import jax.numpy as jnp
from jax import lax
from jax.experimental import pallas as pl
from jax.experimental.pallas import tpu as pltpu
from jax.experimental.pallas import tpu_sc as plsc

N_NODES = 10000
N_EDGES = 160000
N_TRIPLETS = 320000
F = 128
A = 32
R = 8
TB_CUTOFF = 4.0

NC = 2
NS = 16
NW = NC * NS

TRIP_RANGE = 80000
TRIP_ACC_ROWS = 80128
TRIP_DUMP = 80000

ATOM_ACC_ROWS = 10112

TK = 512
EK = 256
T_PAD = 327680

_sds = jax.ShapeDtypeStruct


def _env_poly(r):
    r2 = r * r
    r3 = r2 * r
    return jnp.maximum(1.0 + r3 * (-10.0 + r * (15.0 - 6.0 * r)), 0.0)


def _sigm(x):
    return 0.5 * jnp.tanh(0.5 * x) + 0.5


def _swish(x):
    return x * _sigm(x)


def _sc_mesh():
    return plsc.VectorSubcoreMesh(core_axis_name="c", subcore_axis_name="s",
                                  num_cores=NC, num_subcores=NS)


def _k1_body(af_ref, w_ref, b_ref, out_ref):
    out_ref[...] = _sigm(
        jnp.dot(af_ref[...], w_ref[...], preferred_element_type=jnp.float32)
        + b_ref[...])


def _k1_atomic_filter(af, w_atom, b_atom):
    B = 2000
    return pl.pallas_call(
        _k1_body,
        grid=(N_NODES // B,),
        in_specs=[
            pl.BlockSpec((B, F), lambda i: (i, 0)),
            pl.BlockSpec((F, A), lambda i: (0, 0)),
            pl.BlockSpec((1, A), lambda i: (0, 0)),
        ],
        out_specs=pl.BlockSpec((B, A), lambda i: (i, 0)),
        out_shape=_sds((N_NODES, A), jnp.float32),
    )(af, w_atom, b_atom)


def _k2_body(filt_hbm, dst2_hbm, dist2_hbm, c_hbm,
             idx_v, rows_v, dist_v, env_v, sem):
    c = lax.axis_index("c")
    s = lax.axis_index("s")
    w = c * NS + s

    @pl.loop(w, N_EDGES // EK, step=NW)
    def _(ch):
        pltpu.sync_copy(dst2_hbm.at[pl.ds(ch * 2, 2)], idx_v)
        pltpu.sync_copy(dist2_hbm.at[pl.ds(ch * 2, 2)], dist_v)
        descs = [pltpu.async_copy(filt_hbm.at[idx_v.at[j]],
                                  rows_v.at[pl.ds(j * 128, 128)], sem)
                 for j in range(2)]
        for d in descs:
            d.wait()
        for j in range(2):
            for i in range(8):
                dv = dist_v[j, pl.ds(i * 16, 16)]
                env_v[j, pl.ds(i * 16, 16)] = _env_poly(dv * (1.0 / TB_CUTOFF))

        for j in range(2):
            @pl.loop(0, 8)
            def _(g):
                ev = env_v[j, pl.ds(g * 16, 16)]
                for k in range(16):
                    row = j * 128 + g * 16 + k
                    e = ev[k]
                    rows_v[row, pl.ds(0, 16)] = rows_v[row, pl.ds(0, 16)] * e
                    rows_v[row, pl.ds(16, 16)] = rows_v[row, pl.ds(16, 16)] * e

        pltpu.sync_copy(rows_v, c_hbm.at[pl.ds(ch * EK, EK)])


def _k2_c_rows(filt, dst2, dist2):
    f = functools.partial(
        pl.kernel,
        mesh=_sc_mesh(),
        out_type=_sds((N_EDGES, A), jnp.float32),
        compiler_params=pltpu.CompilerParams(use_tc_tiling_on_sc=False),
        scratch_types=[
            pltpu.VMEM((2, 128), jnp.int32),
            pltpu.VMEM((EK, A), jnp.float32),
            pltpu.VMEM((2, 128), jnp.float32),
            pltpu.VMEM((2, 128), jnp.float32),
            pltpu.SemaphoreType.DMA,
        ],
    )(_k2_body)
    return f(filt, dst2, dist2)


def _k3_body(c_hbm, eij2_hbm, eik2_hbm, ang_hbm, zeros_hbm, out_hbm,
             ij_v, ik_v, loc_v, rows_v, ang_v, sem0, sem1, acc):
    c = lax.axis_index("c")
    s = lax.axis_index("s")
    zh = TRIP_ACC_ROWS // NS
    base_range = c * TRIP_RANGE
    NITER = 40
    sems = (sem0, sem1)

    pltpu.sync_copy(zeros_hbm, acc.at[pl.ds(s * zh, zh)])
    plsc.subcore_barrier()

    def fire(ch, b):
        pltpu.sync_copy(eij2_hbm.at[pl.ds(ch * 4, 4)], ij_v.at[b])
        pltpu.sync_copy(eik2_hbm.at[pl.ds(ch * 4, 4)], ik_v.at[b])
        for j in range(4):
            pltpu.async_copy(c_hbm.at[ik_v.at[b, j]],
                             rows_v.at[b, pl.ds(j * 128, 128)], sems[b])
        pltpu.async_copy(ang_hbm.at[pl.ds(ch * TK, TK)], ang_v.at[b],
                         sems[b])

    def consume(ch, b):
        for j in range(4):
            pltpu.make_async_copy(c_hbm.at[ik_v.at[b, j]],
                                  rows_v.at[b, pl.ds(j * 128, 128)],
                                  sems[b]).wait()
        pltpu.make_async_copy(ang_hbm.at[pl.ds(ch * TK, TK)], ang_v.at[b],
                              sems[b]).wait()
        for j in range(4):
            for i in range(8):
                e = ij_v[b, j, pl.ds(i * 16, 16)]
                l = e - base_range
                ok = (l >= 0) & (l < TRIP_RANGE)
                loc_v[b, j, pl.ds(i * 16, 16)] = jnp.where(
                    ok, l, TRIP_DUMP + (e & 127))

        @plsc.parallel_loop(0, TK, 1, unroll=4)
        def _(i):
            rows_v[b, i, pl.ds(0, 32)] = (rows_v[b, i, pl.ds(0, 32)]
                                          * ang_v[b, i, pl.ds(0, 32)])

        for j in range(4):
            pltpu.async_copy(rows_v.at[b, pl.ds(j * 128, 128)],
                             acc.at[loc_v.at[b, j]], sems[b], add=True)
        for j in range(4):
            pltpu.make_async_copy(rows_v.at[b, pl.ds(j * 128, 128)],
                                  acc.at[loc_v.at[b, j]], sems[b]).wait()

    fire(s, 0)

    @pl.loop(0, NITER // 2)
    def _(kk):
        ch0 = s + 32 * kk
        fire(ch0 + 16, 1)
        consume(ch0, 0)

        @pl.when(kk < NITER // 2 - 1)
        def _():
            fire(ch0 + 32, 0)

        consume(ch0 + 16, 1)

    plsc.subcore_barrier()

    @pl.loop(s, TRIP_RANGE // 80, step=NS)
    def _(ch):
        pltpu.sync_copy(acc.at[pl.ds(ch * 80, 80)],
                        out_hbm.at[pl.ds(base_range + ch * 80, 80)])


def _k3_eft(c_rows, eij2, eik2, angle, zeros3):
    f = functools.partial(
        pl.kernel,
        mesh=_sc_mesh(),
        out_type=_sds((N_EDGES, A), jnp.bfloat16),
        compiler_params=pltpu.CompilerParams(use_tc_tiling_on_sc=False),
        scratch_types=[
            pltpu.VMEM((2, 4, 128), jnp.int32),
            pltpu.VMEM((2, 4, 128), jnp.int32),
            pltpu.VMEM((2, 4, 128), jnp.int32),
            pltpu.VMEM((2, TK, A), jnp.bfloat16),
            pltpu.VMEM((2, TK, A), jnp.bfloat16),
            pltpu.SemaphoreType.DMA,
            pltpu.SemaphoreType.DMA,
            pltpu.VMEM_SHARED((TRIP_ACC_ROWS, A), jnp.bfloat16),
        ],
    )(_k3_body)
    return f(c_rows, eij2, eik2, angle, zeros3)


GK = 128
K4_NITER = 42
E_PAD = GK * NW * K4_NITER


def _k4_body(af_hbm, src2_hbm, dst2_hbm, srcf_hbm, dstf_hbm,
             sidx_v, didx_v, rs_v, rd_v,
             lsem0, lsem1, lsem2, wsem0, wsem1, wsem2):
    c = lax.axis_index("c")
    s = lax.axis_index("s")
    w = c * NS + s
    lsems = (lsem0, lsem1, lsem2)
    wsems = (wsem0, wsem1, wsem2)

    def fire(ch, b):
        pltpu.sync_copy(src2_hbm.at[pl.ds(ch, 1)], sidx_v.at[pl.ds(b, 1)])
        pltpu.sync_copy(dst2_hbm.at[pl.ds(ch, 1)], didx_v.at[pl.ds(b, 1)])
        pltpu.async_copy(af_hbm.at[sidx_v.at[b]], rs_v.at[b], lsems[b])
        pltpu.async_copy(af_hbm.at[didx_v.at[b]], rd_v.at[b], lsems[b])

    def consume(ch, b):
        pltpu.make_async_copy(af_hbm.at[sidx_v.at[b]], rs_v.at[b],
                              lsems[b]).wait()
        pltpu.make_async_copy(af_hbm.at[didx_v.at[b]], rd_v.at[b],
                              lsems[b]).wait()
        pltpu.async_copy(rs_v.at[b], srcf_hbm.at[pl.ds(ch * GK, GK)],
                         wsems[b])
        pltpu.async_copy(rd_v.at[b], dstf_hbm.at[pl.ds(ch * GK, GK)],
                         wsems[b])

    def drain_writes(ch, b):
        pltpu.make_async_copy(rs_v.at[b], srcf_hbm.at[pl.ds(ch * GK, GK)],
                              wsems[b]).wait()
        pltpu.make_async_copy(rd_v.at[b], dstf_hbm.at[pl.ds(ch * GK, GK)],
                              wsems[b]).wait()

    def chk(q):
        return w + NW * q

    fire(chk(0), 0)
    fire(chk(1), 1)

    @pl.loop(0, K4_NITER // 3)
    def _(t):
        k = 3 * t

        @pl.when(t > 0)
        def _():
            drain_writes(chk(k - 1), 2)

        fire(chk(k + 2), 2)
        consume(chk(k), 0)
        consume(chk(k + 1), 1)
        drain_writes(chk(k), 0)

        @pl.when(t < K4_NITER // 3 - 1)
        def _():
            fire(chk(k + 3), 0)

        consume(chk(k + 2), 2)
        drain_writes(chk(k + 1), 1)

        @pl.when(t < K4_NITER // 3 - 1)
        def _():
            fire(chk(k + 4), 1)

    drain_writes(chk(K4_NITER - 1), 2)


def _k4_gather(af, src2p, dst2p):
    f = functools.partial(
        pl.kernel,
        mesh=_sc_mesh(),
        out_type=(_sds((E_PAD, F), jnp.float32),
                  _sds((E_PAD, F), jnp.float32)),
        scratch_types=[
            pltpu.VMEM((3, 128), jnp.int32),
            pltpu.VMEM((3, 128), jnp.int32),
            pltpu.VMEM((3, GK, F), jnp.float32),
            pltpu.VMEM((3, GK, F), jnp.float32),
            pltpu.SemaphoreType.DMA,
            pltpu.SemaphoreType.DMA,
            pltpu.SemaphoreType.DMA,
            pltpu.SemaphoreType.DMA,
            pltpu.SemaphoreType.DMA,
            pltpu.SemaphoreType.DMA,
        ],
    )(_k4_body)
    return f(af, src2p, dst2p)


def _k5_body(eft_ref, dist_ref, ef_ref, s_ref, d_ref, ini_ref,
             wm, bm, wg, bg,
             we1, be1, we2, be2, weg1, beg1, weg2, beg2, wl1, bl1,
             wa1, ba1, wa2, ba2, wag1, bag1, wag2, bag2, wl2, bl2,
             eo_ref, au_ref):
    f32 = jnp.float32
    bf16 = jnp.bfloat16

    def mm(x, w_ref, b_ref):
        return jnp.dot(x, w_ref[...], preferred_element_type=f32) + b_ref[...]

    def mmb(x, w_ref, b_ref):
        return jnp.dot(x.astype(bf16), w_ref[...],
                       preferred_element_type=f32) + b_ref[...]

    env = _env_poly(dist_ref[...] * (1.0 / TB_CUTOFF))
    eft = eft_ref[...].astype(f32) * env
    ef2 = ef_ref[...] + _swish(mm(eft, wm, bm)) * _sigm(mm(eft, wg, bg))

    sf = s_ref[...]
    df = d_ref[...]
    ini = ini_ref[...]

    cat = jnp.concatenate([sf, df, ef2], axis=1)
    m = _swish(mmb(_swish(mmb(cat, we1, be1)), we2, be2))
    g = _sigm(mmb(_swish(mmb(cat, weg1, beg1)), weg2, beg2))
    eo = ef2 + m * g * mm(ini, wl1, bl1)

    cat2 = jnp.concatenate([sf, df, eo], axis=1)
    m2 = _swish(mmb(_swish(mmb(cat2, wa1, ba1)), wa2, ba2))
    g2 = _sigm(mmb(_swish(mmb(cat2, wag1, bag1)), wag2, bag2))
    au = m2 * g2 * mm(ini, wl2, bl2)

    eo_ref[...] = eo
    au_ref[...] = au


def _k5_dense(eft, dist_col, ef, srcf, dstf, ini, p):
    B = 1000
    data_specs = [
        pl.BlockSpec((B, A), lambda i: (i, 0)),
        pl.BlockSpec((B, 1), lambda i: (i, 0)),
        pl.BlockSpec((B, F), lambda i: (i, 0)),
        pl.BlockSpec((B, F), lambda i: (i, 0)),
        pl.BlockSpec((B, F), lambda i: (i, 0)),
        pl.BlockSpec((B, R), lambda i: (i, 0)),
    ]
    bf16 = jnp.bfloat16
    weights = [
        p['W_tb_main'], p['b_tb_main'], p['W_tb_gate'], p['b_tb_gate'],
        p['W_e1'].astype(bf16), p['b_e1'], p['W_e2'].astype(bf16), p['b_e2'],
        p['W_eg1'].astype(bf16), p['b_eg1'],
        p['W_eg2'].astype(bf16), p['b_eg2'],
        p['W_lin1'], p['b_lin1'],
        p['W_a1'].astype(bf16), p['b_a1'], p['W_a2'].astype(bf16), p['b_a2'],
        p['W_ag1'].astype(bf16), p['b_ag1'],
        p['W_ag2'].astype(bf16), p['b_ag2'],
        p['W_lin2'], p['b_lin2'],
    ]
    w_specs = []
    w_in = []
    for w_arr in weights:
        if w_arr.ndim == 1:
            w_arr = w_arr.reshape(1, -1)
        w_in.append(w_arr)
        w_specs.append(pl.BlockSpec(w_arr.shape, lambda i: (0, 0)))
    return pl.pallas_call(
        _k5_body,
        grid=(N_EDGES // B,),
        in_specs=data_specs + w_specs,
        out_specs=(pl.BlockSpec((B, F), lambda i: (i, 0)),
                   pl.BlockSpec((B, F), lambda i: (i, 0))),
        out_shape=(_sds((N_EDGES, F), jnp.float32),
                   _sds((N_EDGES, F), jnp.float32)),
    )(eft, dist_col, ef, srcf, dstf, ini, *w_in)


def _k6_body(au_hbm, src2_hbm, zeros_hbm, p_hbm, idx_v, rows_v, sem, acc):
    c = lax.axis_index("c")
    s = lax.axis_index("s")
    w = c * NS + s
    zr = ATOM_ACC_ROWS // NS

    pltpu.sync_copy(zeros_hbm, acc.at[pl.ds(s * zr, zr)])
    plsc.subcore_barrier()

    @pl.loop(w, N_EDGES // EK, step=NW)
    def _(ch):
        pltpu.sync_copy(src2_hbm.at[pl.ds(ch * 2, 2)], idx_v)
        pltpu.sync_copy(au_hbm.at[pl.ds(ch * EK, EK)], rows_v)
        for j in range(2):
            pltpu.async_copy(rows_v.at[pl.ds(j * 128, 128)],
                             acc.at[idx_v.at[j]], sem, add=True)
        for j in range(2):
            pltpu.make_async_copy(rows_v.at[pl.ds(j * 128, 128)],
                                  acc.at[idx_v.at[j]], sem).wait()

    plsc.subcore_barrier()

    @pl.loop(s, N_NODES // 80, step=NS)
    def _(ch):
        pltpu.sync_copy(acc.at[pl.ds(ch * 80, 80)],
                        p_hbm.at[c, pl.ds(ch * 80, 80)])


def _k6_atom_scatter(au, src2, zeros6):
    f = functools.partial(
        pl.kernel,
        mesh=_sc_mesh(),
        out_type=_sds((NC, N_NODES, F), jnp.float32),
        scratch_types=[
            pltpu.VMEM((2, 128), jnp.int32),
            pltpu.VMEM((EK, F), jnp.float32),
            pltpu.SemaphoreType.DMA,
            pltpu.VMEM_SHARED((ATOM_ACC_ROWS, F), jnp.float32),
        ],
    )(_k6_body)
    return f(au, src2, zeros6)


def _k7_body(af_ref, p0_ref, p1_ref, out_ref):
    out_ref[...] = af_ref[...] + p0_ref[...] + p1_ref[...]


def _k7_final(af, p0, p1):
    B = 2000
    spec = pl.BlockSpec((B, F), lambda i: (i, 0))
    return pl.pallas_call(
        _k7_body,
        grid=(N_NODES // B,),
        in_specs=[spec, spec, spec],
        out_specs=spec,
        out_shape=_sds((N_NODES, F), jnp.float32),
    )(af, p0, p1)


def kernel(atomic_features, edge_features, angle_features,
           initial_edge_features, three_body_indices_with_offset,
           edge_index, edge_dist, params):
    p = params
    src2 = edge_index[0].reshape(N_EDGES // 128, 128)
    dst2 = edge_index[1].reshape(N_EDGES // 128, 128)
    epad = E_PAD - N_EDGES
    src2p = jnp.concatenate(
        [edge_index[0], jnp.zeros((epad,), jnp.int32)]).reshape(
            E_PAD // 128, 128)
    dst2p = jnp.concatenate(
        [edge_index[1], jnp.zeros((epad,), jnp.int32)]).reshape(
            E_PAD // 128, 128)
    npad = T_PAD - N_TRIPLETS
    eij2 = jnp.concatenate(
        [three_body_indices_with_offset[:, 0],
         jnp.full((npad,), 1 << 30, jnp.int32)]).reshape(T_PAD // 128, 128)
    eik2 = jnp.concatenate(
        [three_body_indices_with_offset[:, 1],
         jnp.zeros((npad,), jnp.int32)]).reshape(T_PAD // 128, 128)
    ang_p = jnp.concatenate(
        [angle_features.astype(jnp.bfloat16),
         jnp.zeros((npad, A), jnp.bfloat16)], axis=0)
    dist2_sc = edge_dist.reshape(N_EDGES // 128, 128)
    dist_col = edge_dist.reshape(N_EDGES, 1)

    filt = _k1_atomic_filter(atomic_features, p['W_atom'],
                             p['b_atom'].reshape(1, A))
    c_rows = _k2_c_rows(filt, dst2, dist2_sc)

    zeros3 = jnp.zeros((TRIP_ACC_ROWS // NS, A), jnp.bfloat16)
    eft = _k3_eft(c_rows.astype(jnp.bfloat16), eij2, eik2, ang_p, zeros3)

    srcf, dstf = _k4_gather(atomic_features, src2p, dst2p)

    eo, au = _k5_dense(eft, dist_col, edge_features, srcf, dstf,
                       initial_edge_features, p)

    zeros6 = jnp.zeros((ATOM_ACC_ROWS // NS, F), jnp.float32)
    part = _k6_atom_scatter(au, src2, zeros6)

    atom_out = _k7_final(atomic_features, part[0], part[1])
    return (atom_out, eo)


# speedup vs baseline: 1.6302x; 1.6302x over previous
import functools

import jax
import jax.numpy as jnp
from jax import lax
from jax.experimental import pallas as pl
from jax.experimental.pallas import tpu as pltpu
from jax.experimental.pallas import tpu_sc as plsc

N_NODES = 10000
N_EDGES = 160000
N_TRIPLETS = 320000
F = 128
A = 32
R = 8
TB_CUTOFF = 4.0

NC = 2
NS = 16
NW = NC * NS

TRIP_RANGE = 80000
TRIP_ACC_ROWS = 80128
TRIP_DUMP = 80000

ATOM_ACC_ROWS = 10112

TK = 512
EK = 256
T_PAD = 327680

_sds = jax.ShapeDtypeStruct


def _env_poly(r):
    r2 = r * r
    r3 = r2 * r
    return jnp.maximum(1.0 + r3 * (-10.0 + r * (15.0 - 6.0 * r)), 0.0)


def _sigm(x):
    return 0.5 * jnp.tanh(0.5 * x) + 0.5


def _swish(x):
    return x * _sigm(x)


def _sc_mesh():
    return plsc.VectorSubcoreMesh(core_axis_name="c", subcore_axis_name="s",
                                  num_cores=NC, num_subcores=NS)


def _k1_body(af_ref, w_ref, b_ref, out_ref):
    out_ref[...] = _sigm(
        jnp.dot(af_ref[...], w_ref[...], preferred_element_type=jnp.float32)
        + b_ref[...])


def _k1_atomic_filter(af, w_atom, b_atom):
    B = 2000
    return pl.pallas_call(
        _k1_body,
        grid=(N_NODES // B,),
        in_specs=[
            pl.BlockSpec((B, F), lambda i: (i, 0)),
            pl.BlockSpec((F, A), lambda i: (0, 0)),
            pl.BlockSpec((1, A), lambda i: (0, 0)),
        ],
        out_specs=pl.BlockSpec((B, A), lambda i: (i, 0)),
        out_shape=_sds((N_NODES, A), jnp.float32),
    )(af, w_atom, b_atom)


def _k2_body(filt_hbm, dst2_hbm, dist2_hbm, c_hbm,
             idx_v, rows_v, dist_v, env_v, sem):
    c = lax.axis_index("c")
    s = lax.axis_index("s")
    w = c * NS + s

    @pl.loop(w, N_EDGES // EK, step=NW)
    def _(ch):
        pltpu.sync_copy(dst2_hbm.at[pl.ds(ch * 2, 2)], idx_v)
        pltpu.sync_copy(dist2_hbm.at[pl.ds(ch * 2, 2)], dist_v)
        descs = [pltpu.async_copy(filt_hbm.at[idx_v.at[j]],
                                  rows_v.at[pl.ds(j * 128, 128)], sem)
                 for j in range(2)]
        for d in descs:
            d.wait()
        for j in range(2):
            for i in range(8):
                dv = dist_v[j, pl.ds(i * 16, 16)]
                env_v[j, pl.ds(i * 16, 16)] = _env_poly(dv * (1.0 / TB_CUTOFF))

        for j in range(2):
            @pl.loop(0, 8)
            def _(g):
                ev = env_v[j, pl.ds(g * 16, 16)]
                for k in range(16):
                    row = j * 128 + g * 16 + k
                    e = ev[k]
                    rows_v[row, pl.ds(0, 16)] = rows_v[row, pl.ds(0, 16)] * e
                    rows_v[row, pl.ds(16, 16)] = rows_v[row, pl.ds(16, 16)] * e

        pltpu.sync_copy(rows_v, c_hbm.at[pl.ds(ch * EK, EK)])


def _k2_c_rows(filt, dst2, dist2):
    f = functools.partial(
        pl.kernel,
        mesh=_sc_mesh(),
        out_type=_sds((N_EDGES, A), jnp.float32),
        compiler_params=pltpu.CompilerParams(use_tc_tiling_on_sc=False),
        scratch_types=[
            pltpu.VMEM((2, 128), jnp.int32),
            pltpu.VMEM((EK, A), jnp.float32),
            pltpu.VMEM((2, 128), jnp.float32),
            pltpu.VMEM((2, 128), jnp.float32),
            pltpu.SemaphoreType.DMA,
        ],
    )(_k2_body)
    return f(filt, dst2, dist2)


def _k3_body(c_hbm, eij2_hbm, eik2_hbm, ang_hbm, zeros_hbm, out_hbm,
             ij_v, ik_v, loc_v, rows_v, ang_v, sem0, sem1, acc):
    c = lax.axis_index("c")
    s = lax.axis_index("s")
    zh = TRIP_ACC_ROWS // NS
    base_range = c * TRIP_RANGE
    NITER = 40
    sems = (sem0, sem1)

    pltpu.sync_copy(zeros_hbm, acc.at[pl.ds(s * zh, zh)])
    plsc.subcore_barrier()

    def fire(ch, b):
        pltpu.sync_copy(eij2_hbm.at[pl.ds(ch * 4, 4)], ij_v.at[b])
        pltpu.sync_copy(eik2_hbm.at[pl.ds(ch * 4, 4)], ik_v.at[b])
        for j in range(4):
            pltpu.async_copy(c_hbm.at[ik_v.at[b, j]],
                             rows_v.at[b, pl.ds(j * 128, 128)], sems[b])
        pltpu.async_copy(ang_hbm.at[pl.ds(ch * TK, TK)], ang_v.at[b],
                         sems[b])

    def consume(ch, b):
        for j in range(4):
            pltpu.make_async_copy(c_hbm.at[ik_v.at[b, j]],
                                  rows_v.at[b, pl.ds(j * 128, 128)],
                                  sems[b]).wait()
        pltpu.make_async_copy(ang_hbm.at[pl.ds(ch * TK, TK)], ang_v.at[b],
                              sems[b]).wait()
        for j in range(4):
            for i in range(8):
                e = ij_v[b, j, pl.ds(i * 16, 16)]
                l = e - base_range
                ok = (l >= 0) & (l < TRIP_RANGE)
                loc_v[b, j, pl.ds(i * 16, 16)] = jnp.where(
                    ok, l, TRIP_DUMP + (e & 127))

        @plsc.parallel_loop(0, TK, 1, unroll=4)
        def _(i):
            rows_v[b, i, pl.ds(0, 32)] = (rows_v[b, i, pl.ds(0, 32)]
                                          * ang_v[b, i, pl.ds(0, 32)])

        for j in range(4):
            pltpu.async_copy(rows_v.at[b, pl.ds(j * 128, 128)],
                             acc.at[loc_v.at[b, j]], sems[b], add=True)
        for j in range(4):
            pltpu.make_async_copy(rows_v.at[b, pl.ds(j * 128, 128)],
                                  acc.at[loc_v.at[b, j]], sems[b]).wait()

    fire(s, 0)

    @pl.loop(0, NITER // 2)
    def _(kk):
        ch0 = s + 32 * kk
        fire(ch0 + 16, 1)
        consume(ch0, 0)

        @pl.when(kk < NITER // 2 - 1)
        def _():
            fire(ch0 + 32, 0)

        consume(ch0 + 16, 1)

    plsc.subcore_barrier()

    @pl.loop(s, TRIP_RANGE // 80, step=NS)
    def _(ch):
        pltpu.sync_copy(acc.at[pl.ds(ch * 80, 80)],
                        out_hbm.at[pl.ds(base_range + ch * 80, 80)])


def _k3_eft(c_rows, eij2, eik2, angle, zeros3):
    f = functools.partial(
        pl.kernel,
        mesh=_sc_mesh(),
        out_type=_sds((N_EDGES, A), jnp.bfloat16),
        compiler_params=pltpu.CompilerParams(use_tc_tiling_on_sc=False),
        scratch_types=[
            pltpu.VMEM((2, 4, 128), jnp.int32),
            pltpu.VMEM((2, 4, 128), jnp.int32),
            pltpu.VMEM((2, 4, 128), jnp.int32),
            pltpu.VMEM((2, TK, A), jnp.bfloat16),
            pltpu.VMEM((2, TK, A), jnp.bfloat16),
            pltpu.SemaphoreType.DMA,
            pltpu.SemaphoreType.DMA,
            pltpu.VMEM_SHARED((TRIP_ACC_ROWS, A), jnp.bfloat16),
        ],
    )(_k3_body)
    return f(c_rows, eij2, eik2, angle, zeros3)


def _k4_body(af_hbm, src2_hbm, dst2_hbm, srcf_hbm, dstf_hbm,
             idx_v, rows_v, sem):
    c = lax.axis_index("c")
    s = lax.axis_index("s")
    w = c * NS + s

    @pl.loop(w, N_EDGES // EK, step=NW)
    def _(ch):
        for idx2_hbm, out_hbm in ((src2_hbm, srcf_hbm), (dst2_hbm, dstf_hbm)):
            pltpu.sync_copy(idx2_hbm.at[pl.ds(ch * 2, 2)], idx_v)
            descs = [pltpu.async_copy(af_hbm.at[idx_v.at[j]],
                                      rows_v.at[pl.ds(j * 128, 128)], sem)
                     for j in range(2)]
            for d in descs:
                d.wait()
            pltpu.sync_copy(rows_v, out_hbm.at[pl.ds(ch * EK, EK)])


def _k4_gather(af, src2, dst2):
    f = functools.partial(
        pl.kernel,
        mesh=_sc_mesh(),
        out_type=(_sds((N_EDGES, F), jnp.float32),
                  _sds((N_EDGES, F), jnp.float32)),
        scratch_types=[
            pltpu.VMEM((2, 128), jnp.int32),
            pltpu.VMEM((EK, F), jnp.float32),
            pltpu.SemaphoreType.DMA,
        ],
    )(_k4_body)
    return f(af, src2, dst2)


def _k5_body(eft_ref, dist_ref, ef_ref, s_ref, d_ref, ini_ref,
             wm, bm, wg, bg,
             we1, be1, we2, be2, weg1, beg1, weg2, beg2, wl1, bl1,
             wa1, ba1, wa2, ba2, wag1, bag1, wag2, bag2, wl2, bl2,
             eo_ref, au_ref):
    f32 = jnp.float32
    bf16 = jnp.bfloat16

    def mm(x, w_ref, b_ref):
        return jnp.dot(x, w_ref[...], preferred_element_type=f32) + b_ref[...]

    def mmb(x, w_ref, b_ref):
        return jnp.dot(x.astype(bf16), w_ref[...],
                       preferred_element_type=f32) + b_ref[...]

    env = _env_poly(dist_ref[...] * (1.0 / TB_CUTOFF))
    eft = eft_ref[...].astype(f32) * env
    ef2 = ef_ref[...] + _swish(mm(eft, wm, bm)) * _sigm(mm(eft, wg, bg))

    sf = s_ref[...]
    df = d_ref[...]
    ini = ini_ref[...]

    cat = jnp.concatenate([sf, df, ef2], axis=1)
    m = _swish(mmb(_swish(mmb(cat, we1, be1)), we2, be2))
    g = _sigm(mmb(_swish(mmb(cat, weg1, beg1)), weg2, beg2))
    eo = ef2 + m * g * mm(ini, wl1, bl1)

    cat2 = jnp.concatenate([sf, df, eo], axis=1)
    m2 = _swish(mmb(_swish(mmb(cat2, wa1, ba1)), wa2, ba2))
    g2 = _sigm(mmb(_swish(mmb(cat2, wag1, bag1)), wag2, bag2))
    au = m2 * g2 * mm(ini, wl2, bl2)

    eo_ref[...] = eo
    au_ref[...] = au


def _k5_dense(eft, dist_col, ef, srcf, dstf, ini, p):
    B = 1000
    data_specs = [
        pl.BlockSpec((B, A), lambda i: (i, 0)),
        pl.BlockSpec((B, 1), lambda i: (i, 0)),
        pl.BlockSpec((B, F), lambda i: (i, 0)),
        pl.BlockSpec((B, F), lambda i: (i, 0)),
        pl.BlockSpec((B, F), lambda i: (i, 0)),
        pl.BlockSpec((B, R), lambda i: (i, 0)),
    ]
    bf16 = jnp.bfloat16
    weights = [
        p['W_tb_main'], p['b_tb_main'], p['W_tb_gate'], p['b_tb_gate'],
        p['W_e1'].astype(bf16), p['b_e1'], p['W_e2'].astype(bf16), p['b_e2'],
        p['W_eg1'].astype(bf16), p['b_eg1'],
        p['W_eg2'].astype(bf16), p['b_eg2'],
        p['W_lin1'], p['b_lin1'],
        p['W_a1'].astype(bf16), p['b_a1'], p['W_a2'].astype(bf16), p['b_a2'],
        p['W_ag1'].astype(bf16), p['b_ag1'],
        p['W_ag2'].astype(bf16), p['b_ag2'],
        p['W_lin2'], p['b_lin2'],
    ]
    w_specs = []
    w_in = []
    for w_arr in weights:
        if w_arr.ndim == 1:
            w_arr = w_arr.reshape(1, -1)
        w_in.append(w_arr)
        w_specs.append(pl.BlockSpec(w_arr.shape, lambda i: (0, 0)))
    return pl.pallas_call(
        _k5_body,
        grid=(N_EDGES // B,),
        in_specs=data_specs + w_specs,
        out_specs=(pl.BlockSpec((B, F), lambda i: (i, 0)),
                   pl.BlockSpec((B, F), lambda i: (i, 0))),
        out_shape=(_sds((N_EDGES, F), jnp.float32),
                   _sds((N_EDGES, F), jnp.float32)),
    )(eft, dist_col, ef, srcf, dstf, ini, *w_in)


def _k6_body(au_hbm, src2_hbm, zeros_hbm, p_hbm, idx_v, rows_v, sem, acc):
    c = lax.axis_index("c")
    s = lax.axis_index("s")
    w = c * NS + s
    zr = ATOM_ACC_ROWS // NS

    pltpu.sync_copy(zeros_hbm, acc.at[pl.ds(s * zr, zr)])
    plsc.subcore_barrier()

    @pl.loop(w, N_EDGES // EK, step=NW)
    def _(ch):
        pltpu.sync_copy(src2_hbm.at[pl.ds(ch * 2, 2)], idx_v)
        pltpu.sync_copy(au_hbm.at[pl.ds(ch * EK, EK)], rows_v)
        for j in range(2):
            pltpu.async_copy(rows_v.at[pl.ds(j * 128, 128)],
                             acc.at[idx_v.at[j]], sem, add=True)
        for j in range(2):
            pltpu.make_async_copy(rows_v.at[pl.ds(j * 128, 128)],
                                  acc.at[idx_v.at[j]], sem).wait()

    plsc.subcore_barrier()

    @pl.loop(s, N_NODES // 80, step=NS)
    def _(ch):
        pltpu.sync_copy(acc.at[pl.ds(ch * 80, 80)],
                        p_hbm.at[c, pl.ds(ch * 80, 80)])


def _k6_atom_scatter(au, src2, zeros6):
    f = functools.partial(
        pl.kernel,
        mesh=_sc_mesh(),
        out_type=_sds((NC, N_NODES, F), jnp.float32),
        scratch_types=[
            pltpu.VMEM((2, 128), jnp.int32),
            pltpu.VMEM((EK, F), jnp.float32),
            pltpu.SemaphoreType.DMA,
            pltpu.VMEM_SHARED((ATOM_ACC_ROWS, F), jnp.float32),
        ],
    )(_k6_body)
    return f(au, src2, zeros6)


def _k7_body(af_ref, p0_ref, p1_ref, out_ref):
    out_ref[...] = af_ref[...] + p0_ref[...] + p1_ref[...]


def _k7_final(af, p0, p1):
    B = 2000
    spec = pl.BlockSpec((B, F), lambda i: (i, 0))
    return pl.pallas_call(
        _k7_body,
        grid=(N_NODES // B,),
        in_specs=[spec, spec, spec],
        out_specs=spec,
        out_shape=_sds((N_NODES, F), jnp.float32),
    )(af, p0, p1)


def kernel(atomic_features, edge_features, angle_features,
           initial_edge_features, three_body_indices_with_offset,
           edge_index, edge_dist, params):
    p = params
    src2 = edge_index[0].reshape(N_EDGES // 128, 128)
    dst2 = edge_index[1].reshape(N_EDGES // 128, 128)
    npad = T_PAD - N_TRIPLETS
    eij2 = jnp.concatenate(
        [three_body_indices_with_offset[:, 0],
         jnp.full((npad,), 1 << 30, jnp.int32)]).reshape(T_PAD // 128, 128)
    eik2 = jnp.concatenate(
        [three_body_indices_with_offset[:, 1],
         jnp.zeros((npad,), jnp.int32)]).reshape(T_PAD // 128, 128)
    ang_p = jnp.concatenate(
        [angle_features.astype(jnp.bfloat16),
         jnp.zeros((npad, A), jnp.bfloat16)], axis=0)
    dist2_sc = edge_dist.reshape(N_EDGES // 128, 128)
    dist_col = edge_dist.reshape(N_EDGES, 1)

    filt = _k1_atomic_filter(atomic_features, p['W_atom'],
                             p['b_atom'].reshape(1, A))
    c_rows = _k2_c_rows(filt, dst2, dist2_sc)

    zeros3 = jnp.zeros((TRIP_ACC_ROWS // NS, A), jnp.bfloat16)
    eft = _k3_eft(c_rows.astype(jnp.bfloat16), eij2, eik2, ang_p, zeros3)

    srcf, dstf = _k4_gather(atomic_features, src2, dst2)

    eo, au = _k5_dense(eft, dist_col, edge_features, srcf, dstf,
                       initial_edge_features, p)

    zeros6 = jnp.zeros((ATOM_ACC_ROWS // NS, F), jnp.float32)
    part = _k6_atom_scatter(au, src2, zeros6)

    atom_out = _k7_final(atomic_features, part[0], part[1])
    return (atom_out, eo)


# speedup vs baseline: 1.7380x; 1.0661x over previous
import functools

import jax
import jax.numpy as jnp
from jax import lax
from jax.experimental import pallas as pl
from jax.experimental.pallas import tpu as pltpu
from jax.experimental.pallas import tpu_sc as plsc

N_NODES = 10000
N_EDGES = 160000
N_TRIPLETS = 320000
F = 128
A = 32
R = 8
TB_CUTOFF = 4.0

NC = 2
NS = 16
NW = NC * NS

TRIP_RANGE = 80000
TRIP_ACC_ROWS = 80128
TRIP_DUMP = 80000

ATOM_ACC_ROWS = 10112

TK = 512
EK = 256
T_PAD = 327680

_sds = jax.ShapeDtypeStruct


def _env_poly(r):
    r2 = r * r
    r3 = r2 * r
    return jnp.maximum(1.0 + r3 * (-10.0 + r * (15.0 - 6.0 * r)), 0.0)


def _sigm(x):
    return 0.5 * jnp.tanh(0.5 * x) + 0.5


def _swish(x):
    return x * _sigm(x)


def _sc_mesh():
    return plsc.VectorSubcoreMesh(core_axis_name="c", subcore_axis_name="s",
                                  num_cores=NC, num_subcores=NS)


def _k1_body(af_ref, w_ref, b_ref, out_ref):
    out_ref[...] = _sigm(
        jnp.dot(af_ref[...], w_ref[...], preferred_element_type=jnp.float32)
        + b_ref[...])


def _k1_atomic_filter(af, w_atom, b_atom):
    B = 2000
    return pl.pallas_call(
        _k1_body,
        grid=(N_NODES // B,),
        in_specs=[
            pl.BlockSpec((B, F), lambda i: (i, 0)),
            pl.BlockSpec((F, A), lambda i: (0, 0)),
            pl.BlockSpec((1, A), lambda i: (0, 0)),
        ],
        out_specs=pl.BlockSpec((B, A), lambda i: (i, 0)),
        out_shape=_sds((N_NODES, A), jnp.float32),
    )(af, w_atom, b_atom)


def _k2_body(filt_hbm, dst2_hbm, dist2_hbm, c_hbm,
             idx_v, rows_v, dist_v, env_v, sem):
    c = lax.axis_index("c")
    s = lax.axis_index("s")
    w = c * NS + s

    @pl.loop(w, N_EDGES // EK, step=NW)
    def _(ch):
        pltpu.sync_copy(dst2_hbm.at[pl.ds(ch * 2, 2)], idx_v)
        pltpu.sync_copy(dist2_hbm.at[pl.ds(ch * 2, 2)], dist_v)
        descs = [pltpu.async_copy(filt_hbm.at[idx_v.at[j]],
                                  rows_v.at[pl.ds(j * 128, 128)], sem)
                 for j in range(2)]
        for d in descs:
            d.wait()
        for j in range(2):
            for i in range(8):
                dv = dist_v[j, pl.ds(i * 16, 16)]
                env_v[j, pl.ds(i * 16, 16)] = _env_poly(dv * (1.0 / TB_CUTOFF))

        for j in range(2):
            @pl.loop(0, 8)
            def _(g):
                ev = env_v[j, pl.ds(g * 16, 16)]
                for k in range(16):
                    row = j * 128 + g * 16 + k
                    e = ev[k]
                    rows_v[row, pl.ds(0, 16)] = rows_v[row, pl.ds(0, 16)] * e
                    rows_v[row, pl.ds(16, 16)] = rows_v[row, pl.ds(16, 16)] * e

        pltpu.sync_copy(rows_v, c_hbm.at[pl.ds(ch * EK, EK)])


def _k2_c_rows(filt, dst2, dist2):
    f = functools.partial(
        pl.kernel,
        mesh=_sc_mesh(),
        out_type=_sds((N_EDGES, A), jnp.float32),
        compiler_params=pltpu.CompilerParams(use_tc_tiling_on_sc=False),
        scratch_types=[
            pltpu.VMEM((2, 128), jnp.int32),
            pltpu.VMEM((EK, A), jnp.float32),
            pltpu.VMEM((2, 128), jnp.float32),
            pltpu.VMEM((2, 128), jnp.float32),
            pltpu.SemaphoreType.DMA,
        ],
    )(_k2_body)
    return f(filt, dst2, dist2)


def _k3_body(c_hbm, eij2_hbm, eik2_hbm, ang_hbm, zeros_hbm, out_hbm,
             ij_v, ik_v, loc_v, rows_v, ang_v, sem0, sem1, acc):
    c = lax.axis_index("c")
    s = lax.axis_index("s")
    zh = TRIP_ACC_ROWS // NS
    base_range = c * TRIP_RANGE
    NITER = 40
    sems = (sem0, sem1)

    pltpu.sync_copy(zeros_hbm, acc.at[pl.ds(s * zh, zh)])
    plsc.subcore_barrier()

    def fire(ch, b):
        pltpu.sync_copy(eij2_hbm.at[pl.ds(ch * 4, 4)], ij_v.at[b])
        pltpu.sync_copy(eik2_hbm.at[pl.ds(ch * 4, 4)], ik_v.at[b])
        for j in range(4):
            pltpu.async_copy(c_hbm.at[ik_v.at[b, j]],
                             rows_v.at[b, pl.ds(j * 128, 128)], sems[b])
        pltpu.async_copy(ang_hbm.at[pl.ds(ch * TK, TK)], ang_v.at[b],
                         sems[b])

    def consume(ch, b):
        for j in range(4):
            pltpu.make_async_copy(c_hbm.at[ik_v.at[b, j]],
                                  rows_v.at[b, pl.ds(j * 128, 128)],
                                  sems[b]).wait()
        pltpu.make_async_copy(ang_hbm.at[pl.ds(ch * TK, TK)], ang_v.at[b],
                              sems[b]).wait()
        for j in range(4):
            for i in range(8):
                e = ij_v[b, j, pl.ds(i * 16, 16)]
                l = e - base_range
                ok = (l >= 0) & (l < TRIP_RANGE)
                loc_v[b, j, pl.ds(i * 16, 16)] = jnp.where(
                    ok, l, TRIP_DUMP + (e & 127))

        @plsc.parallel_loop(0, TK, 1, unroll=4)
        def _(i):
            rows_v[b, i, pl.ds(0, 32)] = (rows_v[b, i, pl.ds(0, 32)]
                                          * ang_v[b, i, pl.ds(0, 32)])

        for j in range(4):
            pltpu.async_copy(rows_v.at[b, pl.ds(j * 128, 128)],
                             acc.at[loc_v.at[b, j]], sems[b], add=True)
        for j in range(4):
            pltpu.make_async_copy(rows_v.at[b, pl.ds(j * 128, 128)],
                                  acc.at[loc_v.at[b, j]], sems[b]).wait()

    fire(s, 0)

    @pl.loop(0, NITER // 2)
    def _(kk):
        ch0 = s + 32 * kk
        fire(ch0 + 16, 1)
        consume(ch0, 0)

        @pl.when(kk < NITER // 2 - 1)
        def _():
            fire(ch0 + 32, 0)

        consume(ch0 + 16, 1)

    plsc.subcore_barrier()

    @pl.loop(s, TRIP_RANGE // 80, step=NS)
    def _(ch):
        pltpu.sync_copy(acc.at[pl.ds(ch * 80, 80)],
                        out_hbm.at[pl.ds(base_range + ch * 80, 80)])


def _k3_eft(c_rows, eij2, eik2, angle, zeros3):
    f = functools.partial(
        pl.kernel,
        mesh=_sc_mesh(),
        out_type=_sds((N_EDGES, A), jnp.bfloat16),
        compiler_params=pltpu.CompilerParams(use_tc_tiling_on_sc=False),
        scratch_types=[
            pltpu.VMEM((2, 4, 128), jnp.int32),
            pltpu.VMEM((2, 4, 128), jnp.int32),
            pltpu.VMEM((2, 4, 128), jnp.int32),
            pltpu.VMEM((2, TK, A), jnp.bfloat16),
            pltpu.VMEM((2, TK, A), jnp.bfloat16),
            pltpu.SemaphoreType.DMA,
            pltpu.SemaphoreType.DMA,
            pltpu.VMEM_SHARED((TRIP_ACC_ROWS, A), jnp.bfloat16),
        ],
    )(_k3_body)
    return f(c_rows, eij2, eik2, angle, zeros3)


def _k4_body(af_hbm, src2_hbm, dst2_hbm, srcf_hbm, dstf_hbm,
             sidx_v, didx_v, rs_v, rd_v, sem):
    c = lax.axis_index("c")
    s = lax.axis_index("s")
    w = c * NS + s

    @pl.loop(w, N_EDGES // EK, step=NW)
    def _(ch):
        pltpu.sync_copy(src2_hbm.at[pl.ds(ch * 2, 2)], sidx_v)
        pltpu.sync_copy(dst2_hbm.at[pl.ds(ch * 2, 2)], didx_v)
        descs = []
        for j in range(2):
            descs.append(pltpu.async_copy(
                af_hbm.at[sidx_v.at[j]], rs_v.at[pl.ds(j * 128, 128)], sem))
            descs.append(pltpu.async_copy(
                af_hbm.at[didx_v.at[j]], rd_v.at[pl.ds(j * 128, 128)], sem))
        for d in descs:
            d.wait()
        pltpu.sync_copy(rs_v, srcf_hbm.at[pl.ds(ch * EK, EK)])
        pltpu.sync_copy(rd_v, dstf_hbm.at[pl.ds(ch * EK, EK)])


def _k4_gather(af, src2, dst2):
    f = functools.partial(
        pl.kernel,
        mesh=_sc_mesh(),
        out_type=(_sds((N_EDGES, F), jnp.float32),
                  _sds((N_EDGES, F), jnp.float32)),
        scratch_types=[
            pltpu.VMEM((2, 128), jnp.int32),
            pltpu.VMEM((2, 128), jnp.int32),
            pltpu.VMEM((EK, F), jnp.float32),
            pltpu.VMEM((EK, F), jnp.float32),
            pltpu.SemaphoreType.DMA,
        ],
    )(_k4_body)
    return f(af, src2, dst2)


def _k5_body(eft_ref, dist_ref, ef_ref, s_ref, d_ref, ini_ref,
             wm, bm, wg, bg,
             we1, be1, we2, be2, weg1, beg1, weg2, beg2, wl1, bl1,
             wa1, ba1, wa2, ba2, wag1, bag1, wag2, bag2, wl2, bl2,
             eo_ref, au_ref):
    f32 = jnp.float32
    bf16 = jnp.bfloat16

    def mm(x, w_ref, b_ref):
        return jnp.dot(x, w_ref[...], preferred_element_type=f32) + b_ref[...]

    def mmb(x, w_ref, b_ref):
        return jnp.dot(x.astype(bf16), w_ref[...],
                       preferred_element_type=f32) + b_ref[...]

    env = _env_poly(dist_ref[...] * (1.0 / TB_CUTOFF))
    eft = eft_ref[...].astype(f32) * env
    ef2 = ef_ref[...] + _swish(mm(eft, wm, bm)) * _sigm(mm(eft, wg, bg))

    sf = s_ref[...]
    df = d_ref[...]
    ini = ini_ref[...]

    cat = jnp.concatenate([sf, df, ef2], axis=1)
    m = _swish(mmb(_swish(mmb(cat, we1, be1)), we2, be2))
    g = _sigm(mmb(_swish(mmb(cat, weg1, beg1)), weg2, beg2))
    eo = ef2 + m * g * mm(ini, wl1, bl1)

    cat2 = jnp.concatenate([sf, df, eo], axis=1)
    m2 = _swish(mmb(_swish(mmb(cat2, wa1, ba1)), wa2, ba2))
    g2 = _sigm(mmb(_swish(mmb(cat2, wag1, bag1)), wag2, bag2))
    au = m2 * g2 * mm(ini, wl2, bl2)

    eo_ref[...] = eo
    au_ref[...] = au


def _k5_dense(eft, dist_col, ef, srcf, dstf, ini, p):
    B = 2000
    data_specs = [
        pl.BlockSpec((B, A), lambda i: (i, 0)),
        pl.BlockSpec((B, 1), lambda i: (i, 0)),
        pl.BlockSpec((B, F), lambda i: (i, 0)),
        pl.BlockSpec((B, F), lambda i: (i, 0)),
        pl.BlockSpec((B, F), lambda i: (i, 0)),
        pl.BlockSpec((B, R), lambda i: (i, 0)),
    ]
    bf16 = jnp.bfloat16
    weights = [
        p['W_tb_main'], p['b_tb_main'], p['W_tb_gate'], p['b_tb_gate'],
        p['W_e1'].astype(bf16), p['b_e1'], p['W_e2'].astype(bf16), p['b_e2'],
        p['W_eg1'].astype(bf16), p['b_eg1'],
        p['W_eg2'].astype(bf16), p['b_eg2'],
        p['W_lin1'], p['b_lin1'],
        p['W_a1'].astype(bf16), p['b_a1'], p['W_a2'].astype(bf16), p['b_a2'],
        p['W_ag1'].astype(bf16), p['b_ag1'],
        p['W_ag2'].astype(bf16), p['b_ag2'],
        p['W_lin2'], p['b_lin2'],
    ]
    w_specs = []
    w_in = []
    for w_arr in weights:
        if w_arr.ndim == 1:
            w_arr = w_arr.reshape(1, -1)
        w_in.append(w_arr)
        w_specs.append(pl.BlockSpec(w_arr.shape, lambda i: (0, 0)))
    return pl.pallas_call(
        _k5_body,
        grid=(N_EDGES // B,),
        in_specs=data_specs + w_specs,
        out_specs=(pl.BlockSpec((B, F), lambda i: (i, 0)),
                   pl.BlockSpec((B, F), lambda i: (i, 0))),
        out_shape=(_sds((N_EDGES, F), jnp.float32),
                   _sds((N_EDGES, F), jnp.float32)),
    )(eft, dist_col, ef, srcf, dstf, ini, *w_in)


def _k6_body(au_hbm, src2_hbm, zeros_hbm, p_hbm, idx_v, rows_v, sem, acc):
    c = lax.axis_index("c")
    s = lax.axis_index("s")
    w = c * NS + s
    zr = ATOM_ACC_ROWS // NS

    pltpu.sync_copy(zeros_hbm, acc.at[pl.ds(s * zr, zr)])
    plsc.subcore_barrier()

    @pl.loop(w, N_EDGES // EK, step=NW)
    def _(ch):
        pltpu.sync_copy(src2_hbm.at[pl.ds(ch * 2, 2)], idx_v)
        pltpu.sync_copy(au_hbm.at[pl.ds(ch * EK, EK)], rows_v)
        for j in range(2):
            pltpu.async_copy(rows_v.at[pl.ds(j * 128, 128)],
                             acc.at[idx_v.at[j]], sem, add=True)
        for j in range(2):
            pltpu.make_async_copy(rows_v.at[pl.ds(j * 128, 128)],
                                  acc.at[idx_v.at[j]], sem).wait()

    plsc.subcore_barrier()

    @pl.loop(s, N_NODES // 80, step=NS)
    def _(ch):
        pltpu.sync_copy(acc.at[pl.ds(ch * 80, 80)],
                        p_hbm.at[c, pl.ds(ch * 80, 80)])


def _k6_atom_scatter(au, src2, zeros6):
    f = functools.partial(
        pl.kernel,
        mesh=_sc_mesh(),
        out_type=_sds((NC, N_NODES, F), jnp.float32),
        scratch_types=[
            pltpu.VMEM((2, 128), jnp.int32),
            pltpu.VMEM((EK, F), jnp.float32),
            pltpu.SemaphoreType.DMA,
            pltpu.VMEM_SHARED((ATOM_ACC_ROWS, F), jnp.float32),
        ],
    )(_k6_body)
    return f(au, src2, zeros6)


def _k7_body(af_ref, p0_ref, p1_ref, out_ref):
    out_ref[...] = af_ref[...] + p0_ref[...] + p1_ref[...]


def _k7_final(af, p0, p1):
    B = 2000
    spec = pl.BlockSpec((B, F), lambda i: (i, 0))
    return pl.pallas_call(
        _k7_body,
        grid=(N_NODES // B,),
        in_specs=[spec, spec, spec],
        out_specs=spec,
        out_shape=_sds((N_NODES, F), jnp.float32),
    )(af, p0, p1)


def kernel(atomic_features, edge_features, angle_features,
           initial_edge_features, three_body_indices_with_offset,
           edge_index, edge_dist, params):
    p = params
    src2 = edge_index[0].reshape(N_EDGES // 128, 128)
    dst2 = edge_index[1].reshape(N_EDGES // 128, 128)
    npad = T_PAD - N_TRIPLETS
    eij2 = jnp.concatenate(
        [three_body_indices_with_offset[:, 0],
         jnp.full((npad,), 1 << 30, jnp.int32)]).reshape(T_PAD // 128, 128)
    eik2 = jnp.concatenate(
        [three_body_indices_with_offset[:, 1],
         jnp.zeros((npad,), jnp.int32)]).reshape(T_PAD // 128, 128)
    ang_p = jnp.concatenate(
        [angle_features.astype(jnp.bfloat16),
         jnp.zeros((npad, A), jnp.bfloat16)], axis=0)
    dist2_sc = edge_dist.reshape(N_EDGES // 128, 128)
    dist_col = edge_dist.reshape(N_EDGES, 1)

    filt = _k1_atomic_filter(atomic_features, p['W_atom'],
                             p['b_atom'].reshape(1, A))
    c_rows = _k2_c_rows(filt, dst2, dist2_sc)

    zeros3 = jnp.zeros((TRIP_ACC_ROWS // NS, A), jnp.bfloat16)
    eft = _k3_eft(c_rows.astype(jnp.bfloat16), eij2, eik2, ang_p, zeros3)

    srcf, dstf = _k4_gather(atomic_features, src2, dst2)

    eo, au = _k5_dense(eft, dist_col, edge_features, srcf, dstf,
                       initial_edge_features, p)

    zeros6 = jnp.zeros((ATOM_ACC_ROWS // NS, F), jnp.float32)
    part = _k6_atom_scatter(au, src2, zeros6)

    atom_out = _k7_final(atomic_features, part[0], part[1])
    return (atom_out, eo)


# speedup vs baseline: 1.7627x; 1.0142x over previous
import functools

import jax
import jax.numpy as jnp
from jax import lax
from jax.experimental import pallas as pl
from jax.experimental.pallas import tpu as pltpu
from jax.experimental.pallas import tpu_sc as plsc

N_NODES = 10000
N_EDGES = 160000
N_TRIPLETS = 320000
F = 128
A = 32
R = 8
TB_CUTOFF = 4.0

NC = 2
NS = 16
NW = NC * NS

TRIP_RANGE = 80000
TRIP_ACC_ROWS = 80128
TRIP_DUMP = 80000

ATOM_ACC_ROWS = 10112

TK = 512
EK = 256
T_PAD = 327680

_sds = jax.ShapeDtypeStruct


def _env_poly(r):
    r2 = r * r
    r3 = r2 * r
    return jnp.maximum(1.0 + r3 * (-10.0 + r * (15.0 - 6.0 * r)), 0.0)


def _sigm(x):
    return 0.5 * jnp.tanh(0.5 * x) + 0.5


def _swish(x):
    return x * _sigm(x)


def _sc_mesh():
    return plsc.VectorSubcoreMesh(core_axis_name="c", subcore_axis_name="s",
                                  num_cores=NC, num_subcores=NS)


def _k1_body(af_ref, w_ref, b_ref, out_ref):
    out_ref[...] = _sigm(
        jnp.dot(af_ref[...], w_ref[...], preferred_element_type=jnp.float32)
        + b_ref[...])


def _k1_atomic_filter(af, w_atom, b_atom):
    B = 2000
    return pl.pallas_call(
        _k1_body,
        grid=(N_NODES // B,),
        in_specs=[
            pl.BlockSpec((B, F), lambda i: (i, 0)),
            pl.BlockSpec((F, A), lambda i: (0, 0)),
            pl.BlockSpec((1, A), lambda i: (0, 0)),
        ],
        out_specs=pl.BlockSpec((B, A), lambda i: (i, 0)),
        out_shape=_sds((N_NODES, A), jnp.float32),
    )(af, w_atom, b_atom)


def _k24_body(af_hbm, filt_hbm, src2_hbm, dst2_hbm, dist2_hbm,
              srcf_hbm, dstf_hbm, c_hbm,
              sidx_v, didx_v, rs_v, rd_v, rc_v, dist_v, env_v, sem):
    c = lax.axis_index("c")
    s = lax.axis_index("s")
    w = c * NS + s

    @pl.loop(w, N_EDGES // EK, step=NW)
    def _(ch):
        pltpu.sync_copy(src2_hbm.at[pl.ds(ch * 2, 2)], sidx_v)
        pltpu.sync_copy(dst2_hbm.at[pl.ds(ch * 2, 2)], didx_v)
        descs = []
        for j in range(2):
            descs.append(pltpu.async_copy(
                af_hbm.at[sidx_v.at[j]], rs_v.at[pl.ds(j * 128, 128)], sem))
            descs.append(pltpu.async_copy(
                af_hbm.at[didx_v.at[j]], rd_v.at[pl.ds(j * 128, 128)], sem))
            descs.append(pltpu.async_copy(
                filt_hbm.at[didx_v.at[j]], rc_v.at[pl.ds(j * 128, 128)], sem))
        pltpu.sync_copy(dist2_hbm.at[pl.ds(ch * 2, 2)], dist_v)
        for j in range(2):
            for i in range(8):
                dv = dist_v[j, pl.ds(i * 16, 16)]
                env_v[j, pl.ds(i * 16, 16)] = _env_poly(dv * (1.0 / TB_CUTOFF))
        for d in descs:
            d.wait()
        pltpu.sync_copy(rs_v, srcf_hbm.at[pl.ds(ch * EK, EK)])
        pltpu.sync_copy(rd_v, dstf_hbm.at[pl.ds(ch * EK, EK)])

        for j in range(2):
            @pl.loop(0, 8)
            def _(g):
                ev = env_v[j, pl.ds(g * 16, 16)]
                for k in range(16):
                    row = j * 128 + g * 16 + k
                    e = ev[k]
                    rc_v[row, pl.ds(0, 16)] = rc_v[row, pl.ds(0, 16)] * e
                    rc_v[row, pl.ds(16, 16)] = rc_v[row, pl.ds(16, 16)] * e

        pltpu.sync_copy(rc_v, c_hbm.at[pl.ds(ch * EK, EK)])


def _k24_gather(af, filt, src2, dst2, dist2):
    f = functools.partial(
        pl.kernel,
        mesh=_sc_mesh(),
        out_type=(_sds((N_EDGES, F), jnp.float32),
                  _sds((N_EDGES, F), jnp.float32),
                  _sds((N_EDGES, A), jnp.float32)),
        compiler_params=pltpu.CompilerParams(use_tc_tiling_on_sc=False),
        scratch_types=[
            pltpu.VMEM((2, 128), jnp.int32),
            pltpu.VMEM((2, 128), jnp.int32),
            pltpu.VMEM((EK, F), jnp.float32),
            pltpu.VMEM((EK, F), jnp.float32),
            pltpu.VMEM((EK, A), jnp.float32),
            pltpu.VMEM((2, 128), jnp.float32),
            pltpu.VMEM((2, 128), jnp.float32),
            pltpu.SemaphoreType.DMA,
        ],
    )(_k24_body)
    return f(af, filt, src2, dst2, dist2)


def _k3_body(c_hbm, eij2_hbm, eik2_hbm, ang_hbm, zeros_hbm, out_hbm,
             ij_v, ik_v, loc_v, rows_v, ang_v, sem0, sem1, acc):
    c = lax.axis_index("c")
    s = lax.axis_index("s")
    zh = TRIP_ACC_ROWS // NS
    base_range = c * TRIP_RANGE
    NITER = 40
    sems = (sem0, sem1)

    pltpu.sync_copy(zeros_hbm, acc.at[pl.ds(s * zh, zh)])
    plsc.subcore_barrier()

    def fire(ch, b):
        pltpu.sync_copy(eij2_hbm.at[pl.ds(ch * 4, 4)], ij_v.at[b])
        pltpu.sync_copy(eik2_hbm.at[pl.ds(ch * 4, 4)], ik_v.at[b])
        for j in range(4):
            pltpu.async_copy(c_hbm.at[ik_v.at[b, j]],
                             rows_v.at[b, pl.ds(j * 128, 128)], sems[b])
        pltpu.async_copy(ang_hbm.at[pl.ds(ch * TK, TK)], ang_v.at[b],
                         sems[b])

    def consume(ch, b):
        for j in range(4):
            pltpu.make_async_copy(c_hbm.at[ik_v.at[b, j]],
                                  rows_v.at[b, pl.ds(j * 128, 128)],
                                  sems[b]).wait()
        pltpu.make_async_copy(ang_hbm.at[pl.ds(ch * TK, TK)], ang_v.at[b],
                              sems[b]).wait()
        for j in range(4):
            for i in range(8):
                e = ij_v[b, j, pl.ds(i * 16, 16)]
                l = e - base_range
                ok = (l >= 0) & (l < TRIP_RANGE)
                loc_v[b, j, pl.ds(i * 16, 16)] = jnp.where(
                    ok, l, TRIP_DUMP + (e & 127))

        @plsc.parallel_loop(0, TK, 1, unroll=4)
        def _(i):
            rows_v[b, i, pl.ds(0, 32)] = (rows_v[b, i, pl.ds(0, 32)]
                                          * ang_v[b, i, pl.ds(0, 32)])

        for j in range(4):
            pltpu.async_copy(rows_v.at[b, pl.ds(j * 128, 128)],
                             acc.at[loc_v.at[b, j]], sems[b], add=True)
        for j in range(4):
            pltpu.make_async_copy(rows_v.at[b, pl.ds(j * 128, 128)],
                                  acc.at[loc_v.at[b, j]], sems[b]).wait()

    fire(s, 0)

    @pl.loop(0, NITER // 2)
    def _(kk):
        ch0 = s + 32 * kk
        fire(ch0 + 16, 1)
        consume(ch0, 0)

        @pl.when(kk < NITER // 2 - 1)
        def _():
            fire(ch0 + 32, 0)

        consume(ch0 + 16, 1)

    plsc.subcore_barrier()

    @pl.loop(s, TRIP_RANGE // 80, step=NS)
    def _(ch):
        pltpu.sync_copy(acc.at[pl.ds(ch * 80, 80)],
                        out_hbm.at[pl.ds(base_range + ch * 80, 80)])


def _k3_eft(c_rows, eij2, eik2, angle, zeros3):
    f = functools.partial(
        pl.kernel,
        mesh=_sc_mesh(),
        out_type=_sds((N_EDGES, A), jnp.bfloat16),
        compiler_params=pltpu.CompilerParams(use_tc_tiling_on_sc=False),
        scratch_types=[
            pltpu.VMEM((2, 4, 128), jnp.int32),
            pltpu.VMEM((2, 4, 128), jnp.int32),
            pltpu.VMEM((2, 4, 128), jnp.int32),
            pltpu.VMEM((2, TK, A), jnp.bfloat16),
            pltpu.VMEM((2, TK, A), jnp.bfloat16),
            pltpu.SemaphoreType.DMA,
            pltpu.SemaphoreType.DMA,
            pltpu.VMEM_SHARED((TRIP_ACC_ROWS, A), jnp.bfloat16),
        ],
    )(_k3_body)
    return f(c_rows, eij2, eik2, angle, zeros3)


def _k5_body(eft_ref, dist_ref, ef_ref, s_ref, d_ref, ini_ref,
             wm, bm, wg, bg,
             we1, be1, we2, be2, weg1, beg1, weg2, beg2, wl1, bl1,
             wa1, ba1, wa2, ba2, wag1, bag1, wag2, bag2, wl2, bl2,
             eo_ref, au_ref):
    f32 = jnp.float32
    bf16 = jnp.bfloat16

    def mm(x, w_ref, b_ref):
        return jnp.dot(x, w_ref[...], preferred_element_type=f32) + b_ref[...]

    def mmb(x, w_ref, b_ref):
        return jnp.dot(x.astype(bf16), w_ref[...],
                       preferred_element_type=f32) + b_ref[...]

    env = _env_poly(dist_ref[...] * (1.0 / TB_CUTOFF))
    eft = eft_ref[...].astype(f32) * env
    ef2 = ef_ref[...] + _swish(mm(eft, wm, bm)) * _sigm(mm(eft, wg, bg))

    sf = s_ref[...]
    df = d_ref[...]
    ini = ini_ref[...]

    cat = jnp.concatenate([sf, df, ef2], axis=1)
    m = _swish(mmb(_swish(mmb(cat, we1, be1)), we2, be2))
    g = _sigm(mmb(_swish(mmb(cat, weg1, beg1)), weg2, beg2))
    eo = ef2 + m * g * mm(ini, wl1, bl1)

    cat2 = jnp.concatenate([sf, df, eo], axis=1)
    m2 = _swish(mmb(_swish(mmb(cat2, wa1, ba1)), wa2, ba2))
    g2 = _sigm(mmb(_swish(mmb(cat2, wag1, bag1)), wag2, bag2))
    au = m2 * g2 * mm(ini, wl2, bl2)

    eo_ref[...] = eo
    au_ref[...] = au


def _k5_dense(eft, dist_col, ef, srcf, dstf, ini, p):
    B = 2000
    data_specs = [
        pl.BlockSpec((B, A), lambda i: (i, 0)),
        pl.BlockSpec((B, 1), lambda i: (i, 0)),
        pl.BlockSpec((B, F), lambda i: (i, 0)),
        pl.BlockSpec((B, F), lambda i: (i, 0)),
        pl.BlockSpec((B, F), lambda i: (i, 0)),
        pl.BlockSpec((B, R), lambda i: (i, 0)),
    ]
    bf16 = jnp.bfloat16
    weights = [
        p['W_tb_main'], p['b_tb_main'], p['W_tb_gate'], p['b_tb_gate'],
        p['W_e1'].astype(bf16), p['b_e1'], p['W_e2'].astype(bf16), p['b_e2'],
        p['W_eg1'].astype(bf16), p['b_eg1'],
        p['W_eg2'].astype(bf16), p['b_eg2'],
        p['W_lin1'], p['b_lin1'],
        p['W_a1'].astype(bf16), p['b_a1'], p['W_a2'].astype(bf16), p['b_a2'],
        p['W_ag1'].astype(bf16), p['b_ag1'],
        p['W_ag2'].astype(bf16), p['b_ag2'],
        p['W_lin2'], p['b_lin2'],
    ]
    w_specs = []
    w_in = []
    for w_arr in weights:
        if w_arr.ndim == 1:
            w_arr = w_arr.reshape(1, -1)
        w_in.append(w_arr)
        w_specs.append(pl.BlockSpec(w_arr.shape, lambda i: (0, 0)))
    return pl.pallas_call(
        _k5_body,
        grid=(N_EDGES // B,),
        in_specs=data_specs + w_specs,
        out_specs=(pl.BlockSpec((B, F), lambda i: (i, 0)),
                   pl.BlockSpec((B, F), lambda i: (i, 0))),
        out_shape=(_sds((N_EDGES, F), jnp.float32),
                   _sds((N_EDGES, F), jnp.float32)),
    )(eft, dist_col, ef, srcf, dstf, ini, *w_in)


def _k6_body(au_hbm, src2_hbm, zeros_hbm, p_hbm, idx_v, rows_v, sem, acc):
    c = lax.axis_index("c")
    s = lax.axis_index("s")
    w = c * NS + s
    zr = ATOM_ACC_ROWS // NS

    pltpu.sync_copy(zeros_hbm, acc.at[pl.ds(s * zr, zr)])
    plsc.subcore_barrier()

    @pl.loop(w, N_EDGES // EK, step=NW)
    def _(ch):
        pltpu.sync_copy(src2_hbm.at[pl.ds(ch * 2, 2)], idx_v)
        pltpu.sync_copy(au_hbm.at[pl.ds(ch * EK, EK)], rows_v)
        for j in range(2):
            pltpu.async_copy(rows_v.at[pl.ds(j * 128, 128)],
                             acc.at[idx_v.at[j]], sem, add=True)
        for j in range(2):
            pltpu.make_async_copy(rows_v.at[pl.ds(j * 128, 128)],
                                  acc.at[idx_v.at[j]], sem).wait()

    plsc.subcore_barrier()

    @pl.loop(s, N_NODES // 80, step=NS)
    def _(ch):
        pltpu.sync_copy(acc.at[pl.ds(ch * 80, 80)],
                        p_hbm.at[c, pl.ds(ch * 80, 80)])


def _k6_atom_scatter(au, src2, zeros6):
    f = functools.partial(
        pl.kernel,
        mesh=_sc_mesh(),
        out_type=_sds((NC, N_NODES, F), jnp.float32),
        scratch_types=[
            pltpu.VMEM((2, 128), jnp.int32),
            pltpu.VMEM((EK, F), jnp.float32),
            pltpu.SemaphoreType.DMA,
            pltpu.VMEM_SHARED((ATOM_ACC_ROWS, F), jnp.float32),
        ],
    )(_k6_body)
    return f(au, src2, zeros6)


def _k7_body(af_ref, p0_ref, p1_ref, out_ref):
    out_ref[...] = af_ref[...] + p0_ref[...] + p1_ref[...]


def _k7_final(af, p0, p1):
    B = 2000
    spec = pl.BlockSpec((B, F), lambda i: (i, 0))
    return pl.pallas_call(
        _k7_body,
        grid=(N_NODES // B,),
        in_specs=[spec, spec, spec],
        out_specs=spec,
        out_shape=_sds((N_NODES, F), jnp.float32),
    )(af, p0, p1)


def kernel(atomic_features, edge_features, angle_features,
           initial_edge_features, three_body_indices_with_offset,
           edge_index, edge_dist, params):
    p = params
    src2 = edge_index[0].reshape(N_EDGES // 128, 128)
    dst2 = edge_index[1].reshape(N_EDGES // 128, 128)
    npad = T_PAD - N_TRIPLETS
    eij2 = jnp.concatenate(
        [three_body_indices_with_offset[:, 0],
         jnp.full((npad,), 1 << 30, jnp.int32)]).reshape(T_PAD // 128, 128)
    eik2 = jnp.concatenate(
        [three_body_indices_with_offset[:, 1],
         jnp.zeros((npad,), jnp.int32)]).reshape(T_PAD // 128, 128)
    ang_p = jnp.concatenate(
        [angle_features.astype(jnp.bfloat16),
         jnp.zeros((npad, A), jnp.bfloat16)], axis=0)
    dist2_sc = edge_dist.reshape(N_EDGES // 128, 128)
    dist_col = edge_dist.reshape(N_EDGES, 1)

    filt = _k1_atomic_filter(atomic_features, p['W_atom'],
                             p['b_atom'].reshape(1, A))
    srcf, dstf, c_rows = _k24_gather(atomic_features, filt, src2, dst2,
                                     dist2_sc)

    zeros3 = jnp.zeros((TRIP_ACC_ROWS // NS, A), jnp.bfloat16)
    eft = _k3_eft(c_rows.astype(jnp.bfloat16), eij2, eik2, ang_p, zeros3)

    eo, au = _k5_dense(eft, dist_col, edge_features, srcf, dstf,
                       initial_edge_features, p)

    zeros6 = jnp.zeros((ATOM_ACC_ROWS // NS, F), jnp.float32)
    part = _k6_atom_scatter(au, src2, zeros6)

    atom_out = _k7_final(atomic_features, part[0], part[1])
    return (atom_out, eo)


# speedup vs baseline: 1.7732x; 1.0060x over previous
import functools

import jax
import jax.numpy as jnp
from jax import lax
from jax.experimental import pallas as pl
from jax.experimental.pallas import tpu as pltpu
from jax.experimental.pallas import tpu_sc as plsc

N_NODES = 10000
N_EDGES = 160000
N_TRIPLETS = 320000
F = 128
A = 32
R = 8
TB_CUTOFF = 4.0

NC = 2
NS = 16
NW = NC * NS

TRIP_RANGE = 80000
TRIP_ACC_ROWS = 80128
TRIP_DUMP = 80000

ATOM_ACC_ROWS = 10112

TK = 512
EK = 256
T_PAD = 327680

_sds = jax.ShapeDtypeStruct


def _env_poly(r):
    r2 = r * r
    r3 = r2 * r
    return jnp.maximum(1.0 + r3 * (-10.0 + r * (15.0 - 6.0 * r)), 0.0)


def _sigm(x):
    return 0.5 * jnp.tanh(0.5 * x) + 0.5


def _swish(x):
    return x * _sigm(x)


def _sc_mesh():
    return plsc.VectorSubcoreMesh(core_axis_name="c", subcore_axis_name="s",
                                  num_cores=NC, num_subcores=NS)


def _k1_body(af_ref, w_ref, b_ref, out_ref):
    out_ref[...] = _sigm(
        jnp.dot(af_ref[...], w_ref[...], preferred_element_type=jnp.float32)
        + b_ref[...])


def _k1_atomic_filter(af, w_atom, b_atom):
    B = 2000
    return pl.pallas_call(
        _k1_body,
        grid=(N_NODES // B,),
        in_specs=[
            pl.BlockSpec((B, F), lambda i: (i, 0)),
            pl.BlockSpec((F, A), lambda i: (0, 0)),
            pl.BlockSpec((1, A), lambda i: (0, 0)),
        ],
        out_specs=pl.BlockSpec((B, A), lambda i: (i, 0)),
        out_shape=_sds((N_NODES, A), jnp.float32),
    )(af, w_atom, b_atom)


def _k24_body(af_hbm, filt_hbm, src2_hbm, dst2_hbm, dist2_hbm,
              srcf_hbm, dstf_hbm, c_hbm,
              sidx_v, didx_v, rs_v, rd_v, rc_v, dist_v, env_v, sem):
    c = lax.axis_index("c")
    s = lax.axis_index("s")
    w = c * NS + s

    @pl.loop(w, N_EDGES // EK, step=NW)
    def _(ch):
        pltpu.sync_copy(src2_hbm.at[pl.ds(ch * 2, 2)], sidx_v)
        pltpu.sync_copy(dst2_hbm.at[pl.ds(ch * 2, 2)], didx_v)
        descs = []
        for j in range(2):
            descs.append(pltpu.async_copy(
                af_hbm.at[sidx_v.at[j]], rs_v.at[pl.ds(j * 128, 128)], sem))
            descs.append(pltpu.async_copy(
                af_hbm.at[didx_v.at[j]], rd_v.at[pl.ds(j * 128, 128)], sem))
            descs.append(pltpu.async_copy(
                filt_hbm.at[didx_v.at[j]], rc_v.at[pl.ds(j * 128, 128)], sem))
        pltpu.sync_copy(dist2_hbm.at[pl.ds(ch * 2, 2)], dist_v)
        for j in range(2):
            for i in range(8):
                dv = dist_v[j, pl.ds(i * 16, 16)]
                env_v[j, pl.ds(i * 16, 16)] = _env_poly(dv * (1.0 / TB_CUTOFF))
        for d in descs:
            d.wait()
        pltpu.sync_copy(rs_v, srcf_hbm.at[pl.ds(ch * EK, EK)])
        pltpu.sync_copy(rd_v, dstf_hbm.at[pl.ds(ch * EK, EK)])

        for j in range(2):
            @pl.loop(0, 8)
            def _(g):
                ev = env_v[j, pl.ds(g * 16, 16)]
                for k in range(16):
                    row = j * 128 + g * 16 + k
                    e = ev[k]
                    rc_v[row, pl.ds(0, 16)] = rc_v[row, pl.ds(0, 16)] * e
                    rc_v[row, pl.ds(16, 16)] = rc_v[row, pl.ds(16, 16)] * e

        pltpu.sync_copy(rc_v, c_hbm.at[pl.ds(ch * EK, EK)])


def _k24_gather(af, filt, src2, dst2, dist2):
    f = functools.partial(
        pl.kernel,
        mesh=_sc_mesh(),
        out_type=(_sds((N_EDGES, F), jnp.float32),
                  _sds((N_EDGES, F), jnp.float32),
                  _sds((N_EDGES, A), jnp.float32)),
        compiler_params=pltpu.CompilerParams(use_tc_tiling_on_sc=False),
        scratch_types=[
            pltpu.VMEM((2, 128), jnp.int32),
            pltpu.VMEM((2, 128), jnp.int32),
            pltpu.VMEM((EK, F), jnp.float32),
            pltpu.VMEM((EK, F), jnp.float32),
            pltpu.VMEM((EK, A), jnp.float32),
            pltpu.VMEM((2, 128), jnp.float32),
            pltpu.VMEM((2, 128), jnp.float32),
            pltpu.SemaphoreType.DMA,
        ],
    )(_k24_body)
    return f(af, filt, src2, dst2, dist2)


def _k3_body(c_hbm, ijk2_hbm, ang_hbm, zeros_hbm, out_hbm,
             ijk_v, loc_v, rows_v, ang_v, sem0, sem1, acc):
    c = lax.axis_index("c")
    s = lax.axis_index("s")
    zh = TRIP_ACC_ROWS // NS
    base_range = c * TRIP_RANGE
    NITER = 40
    sems = (sem0, sem1)

    pltpu.sync_copy(zeros_hbm, acc.at[pl.ds(s * zh, zh)])
    plsc.subcore_barrier()

    def fire(ch, b):
        pltpu.sync_copy(ijk2_hbm.at[pl.ds(ch * 8, 8)], ijk_v.at[b])
        for j in range(4):
            pltpu.async_copy(c_hbm.at[ijk_v.at[b, 4 + j]],
                             rows_v.at[b, pl.ds(j * 128, 128)], sems[b])
        pltpu.async_copy(ang_hbm.at[pl.ds(ch * TK, TK)], ang_v.at[b],
                         sems[b])

    def consume(ch, b):
        for j in range(4):
            pltpu.make_async_copy(c_hbm.at[ijk_v.at[b, 4 + j]],
                                  rows_v.at[b, pl.ds(j * 128, 128)],
                                  sems[b]).wait()
        pltpu.make_async_copy(ang_hbm.at[pl.ds(ch * TK, TK)], ang_v.at[b],
                              sems[b]).wait()
        for j in range(4):
            for i in range(8):
                e = ijk_v[b, j, pl.ds(i * 16, 16)]
                l = e - base_range
                ok = (l >= 0) & (l < TRIP_RANGE)
                loc_v[b, j, pl.ds(i * 16, 16)] = jnp.where(
                    ok, l, TRIP_DUMP + (e & 127))

        @plsc.parallel_loop(0, TK, 1, unroll=4)
        def _(i):
            rows_v[b, i, pl.ds(0, 32)] = (rows_v[b, i, pl.ds(0, 32)]
                                          * ang_v[b, i, pl.ds(0, 32)])

        for j in range(4):
            pltpu.async_copy(rows_v.at[b, pl.ds(j * 128, 128)],
                             acc.at[loc_v.at[b, j]], sems[b], add=True)
        for j in range(4):
            pltpu.make_async_copy(rows_v.at[b, pl.ds(j * 128, 128)],
                                  acc.at[loc_v.at[b, j]], sems[b]).wait()

    fire(s, 0)

    @pl.loop(0, NITER // 2)
    def _(kk):
        ch0 = s + 32 * kk
        fire(ch0 + 16, 1)
        consume(ch0, 0)

        @pl.when(kk < NITER // 2 - 1)
        def _():
            fire(ch0 + 32, 0)

        consume(ch0 + 16, 1)

    plsc.subcore_barrier()

    @pl.loop(s, TRIP_RANGE // 80, step=NS)
    def _(ch):
        pltpu.sync_copy(acc.at[pl.ds(ch * 80, 80)],
                        out_hbm.at[pl.ds(base_range + ch * 80, 80)])


def _k3_eft(c_rows, ijk2, angle, zeros3):
    f = functools.partial(
        pl.kernel,
        mesh=_sc_mesh(),
        out_type=_sds((N_EDGES, A), jnp.bfloat16),
        compiler_params=pltpu.CompilerParams(use_tc_tiling_on_sc=False),
        scratch_types=[
            pltpu.VMEM((2, 8, 128), jnp.int32),
            pltpu.VMEM((2, 4, 128), jnp.int32),
            pltpu.VMEM((2, TK, A), jnp.bfloat16),
            pltpu.VMEM((2, TK, A), jnp.bfloat16),
            pltpu.SemaphoreType.DMA,
            pltpu.SemaphoreType.DMA,
            pltpu.VMEM_SHARED((TRIP_ACC_ROWS, A), jnp.bfloat16),
        ],
    )(_k3_body)
    return f(c_rows, ijk2, angle, zeros3)


def _k5_body(eft_ref, dist_ref, ef_ref, s_ref, d_ref, ini_ref,
             wm, bm, wg, bg,
             we1, be1, we2, be2, weg1, beg1, weg2, beg2, wl1, bl1,
             wa1, ba1, wa2, ba2, wag1, bag1, wag2, bag2, wl2, bl2,
             eo_ref, au_ref):
    f32 = jnp.float32
    bf16 = jnp.bfloat16

    def mm(x, w_ref, b_ref):
        return jnp.dot(x, w_ref[...], preferred_element_type=f32) + b_ref[...]

    def mmb(x, w_ref, b_ref):
        return jnp.dot(x.astype(bf16), w_ref[...],
                       preferred_element_type=f32) + b_ref[...]

    env = _env_poly(dist_ref[...] * (1.0 / TB_CUTOFF))
    eft = eft_ref[...].astype(f32) * env
    ef2 = ef_ref[...] + _swish(mm(eft, wm, bm)) * _sigm(mm(eft, wg, bg))

    sf = s_ref[...]
    df = d_ref[...]
    ini = ini_ref[...]

    cat = jnp.concatenate([sf, df, ef2], axis=1)
    m = _swish(mmb(_swish(mmb(cat, we1, be1)), we2, be2))
    g = _sigm(mmb(_swish(mmb(cat, weg1, beg1)), weg2, beg2))
    eo = ef2 + m * g * mm(ini, wl1, bl1)

    cat2 = jnp.concatenate([sf, df, eo], axis=1)
    m2 = _swish(mmb(_swish(mmb(cat2, wa1, ba1)), wa2, ba2))
    g2 = _sigm(mmb(_swish(mmb(cat2, wag1, bag1)), wag2, bag2))
    au = m2 * g2 * mm(ini, wl2, bl2)

    eo_ref[...] = eo
    au_ref[...] = au


def _k5_dense(eft, dist_col, ef, srcf, dstf, ini, p):
    B = 2000
    data_specs = [
        pl.BlockSpec((B, A), lambda i: (i, 0)),
        pl.BlockSpec((B, 1), lambda i: (i, 0)),
        pl.BlockSpec((B, F), lambda i: (i, 0)),
        pl.BlockSpec((B, F), lambda i: (i, 0)),
        pl.BlockSpec((B, F), lambda i: (i, 0)),
        pl.BlockSpec((B, R), lambda i: (i, 0)),
    ]
    bf16 = jnp.bfloat16
    weights = [
        p['W_tb_main'], p['b_tb_main'], p['W_tb_gate'], p['b_tb_gate'],
        p['W_e1'].astype(bf16), p['b_e1'], p['W_e2'].astype(bf16), p['b_e2'],
        p['W_eg1'].astype(bf16), p['b_eg1'],
        p['W_eg2'].astype(bf16), p['b_eg2'],
        p['W_lin1'], p['b_lin1'],
        p['W_a1'].astype(bf16), p['b_a1'], p['W_a2'].astype(bf16), p['b_a2'],
        p['W_ag1'].astype(bf16), p['b_ag1'],
        p['W_ag2'].astype(bf16), p['b_ag2'],
        p['W_lin2'], p['b_lin2'],
    ]
    w_specs = []
    w_in = []
    for w_arr in weights:
        if w_arr.ndim == 1:
            w_arr = w_arr.reshape(1, -1)
        w_in.append(w_arr)
        w_specs.append(pl.BlockSpec(w_arr.shape, lambda i: (0, 0)))
    return pl.pallas_call(
        _k5_body,
        grid=(N_EDGES // B,),
        in_specs=data_specs + w_specs,
        out_specs=(pl.BlockSpec((B, F), lambda i: (i, 0)),
                   pl.BlockSpec((B, F), lambda i: (i, 0))),
        out_shape=(_sds((N_EDGES, F), jnp.float32),
                   _sds((N_EDGES, F), jnp.float32)),
    )(eft, dist_col, ef, srcf, dstf, ini, *w_in)


def _k6_body(au_hbm, src2_hbm, zeros_hbm, p_hbm, idx_v, rows_v, sem, acc):
    c = lax.axis_index("c")
    s = lax.axis_index("s")
    w = c * NS + s
    zr = ATOM_ACC_ROWS // NS

    pltpu.sync_copy(zeros_hbm, acc.at[pl.ds(s * zr, zr)])
    plsc.subcore_barrier()

    @pl.loop(w, N_EDGES // EK, step=NW)
    def _(ch):
        pltpu.sync_copy(src2_hbm.at[pl.ds(ch * 2, 2)], idx_v)
        pltpu.sync_copy(au_hbm.at[pl.ds(ch * EK, EK)], rows_v)
        for j in range(2):
            pltpu.async_copy(rows_v.at[pl.ds(j * 128, 128)],
                             acc.at[idx_v.at[j]], sem, add=True)
        for j in range(2):
            pltpu.make_async_copy(rows_v.at[pl.ds(j * 128, 128)],
                                  acc.at[idx_v.at[j]], sem).wait()

    plsc.subcore_barrier()

    @pl.loop(s, N_NODES // 80, step=NS)
    def _(ch):
        pltpu.sync_copy(acc.at[pl.ds(ch * 80, 80)],
                        p_hbm.at[c, pl.ds(ch * 80, 80)])


def _k6_atom_scatter(au, src2, zeros6):
    f = functools.partial(
        pl.kernel,
        mesh=_sc_mesh(),
        out_type=_sds((NC, N_NODES, F), jnp.float32),
        scratch_types=[
            pltpu.VMEM((2, 128), jnp.int32),
            pltpu.VMEM((EK, F), jnp.float32),
            pltpu.SemaphoreType.DMA,
            pltpu.VMEM_SHARED((ATOM_ACC_ROWS, F), jnp.float32),
        ],
    )(_k6_body)
    return f(au, src2, zeros6)


def _k7_body(af_ref, p0_ref, p1_ref, out_ref):
    out_ref[...] = af_ref[...] + p0_ref[...] + p1_ref[...]


def _k7_final(af, p0, p1):
    B = 2000
    spec = pl.BlockSpec((B, F), lambda i: (i, 0))
    return pl.pallas_call(
        _k7_body,
        grid=(N_NODES // B,),
        in_specs=[spec, spec, spec],
        out_specs=spec,
        out_shape=_sds((N_NODES, F), jnp.float32),
    )(af, p0, p1)


def kernel(atomic_features, edge_features, angle_features,
           initial_edge_features, three_body_indices_with_offset,
           edge_index, edge_dist, params):
    p = params
    src2 = edge_index[0].reshape(N_EDGES // 128, 128)
    dst2 = edge_index[1].reshape(N_EDGES // 128, 128)
    npad = T_PAD - N_TRIPLETS
    eij3 = jnp.concatenate(
        [three_body_indices_with_offset[:, 0],
         jnp.full((npad,), 1 << 30, jnp.int32)]).reshape(T_PAD // TK, 4, 128)
    eik3 = jnp.concatenate(
        [three_body_indices_with_offset[:, 1],
         jnp.zeros((npad,), jnp.int32)]).reshape(T_PAD // TK, 4, 128)
    ijk2 = jnp.concatenate([eij3, eik3], axis=1).reshape(-1, 128)
    ang_p = jnp.concatenate(
        [angle_features.astype(jnp.bfloat16),
         jnp.zeros((npad, A), jnp.bfloat16)], axis=0)
    dist2_sc = edge_dist.reshape(N_EDGES // 128, 128)
    dist_col = edge_dist.reshape(N_EDGES, 1)

    filt = _k1_atomic_filter(atomic_features, p['W_atom'],
                             p['b_atom'].reshape(1, A))
    srcf, dstf, c_rows = _k24_gather(atomic_features, filt, src2, dst2,
                                     dist2_sc)

    zeros3 = jnp.zeros((TRIP_ACC_ROWS // NS, A), jnp.bfloat16)
    eft = _k3_eft(c_rows.astype(jnp.bfloat16), ijk2, ang_p, zeros3)

    eo, au = _k5_dense(eft, dist_col, edge_features, srcf, dstf,
                       initial_edge_features, p)

    zeros6 = jnp.zeros((ATOM_ACC_ROWS // NS, F), jnp.float32)
    part = _k6_atom_scatter(au, src2, zeros6)

    atom_out = _k7_final(atomic_features, part[0], part[1])
    return (atom_out, eo)
